# baseline (speedup 1.0000x reference)
import functools

import jax
import jax.numpy as jnp
from jax import lax
from jax.experimental import pallas as pl
from jax.experimental.pallas import tpu as pltpu
from jax.experimental.pallas import tpu_sc as plsc

N_NODES = 50000
K_MAX = 8
INPUT_BITS = 512
BATCH = 128
N_OUT = 10

NW = 32
NPAD = 50176
NT = NPAD // NW
BLK = 512
NBLK = NPAD // BLK
CH = 16
NCHUNK = NT // CH
ZROW = N_NODES


def _prep_body(states_ref, x_ref, wcol_ref, lut_ref, wp_ref, pert_ref, lutw_ref):
    i = pl.program_id(0)
    base = i * BLK
    wcol = wcol_ref[...]
    jcol = base + lax.broadcasted_iota(jnp.int32, (BLK, BLK), 1)
    m = (wcol == jcol).astype(jnp.float32)
    s = jnp.dot(x_ref[...], m, preferred_element_type=jnp.float32)
    pert_ref[...] = states_ref[...] ^ s.astype(jnp.int32)
    lut_f = lut_ref[...].astype(jnp.float32)
    words = jnp.dot(lut_f, wp_ref[...], preferred_element_type=jnp.float32)
    lutw_ref[...] = words.astype(jnp.int32)


def _prep(states_p, x_f, w_col, lut_p, wp):
    return pl.pallas_call(
        _prep_body,
        grid=(NBLK,),
        in_specs=[
            pl.BlockSpec((BATCH, BLK), lambda i: (0, i)),
            pl.BlockSpec((BATCH, INPUT_BITS), lambda i: (0, 0)),
            pl.BlockSpec((INPUT_BITS, 1), lambda i: (0, 0)),
            pl.BlockSpec((BLK, 256), lambda i: (i, 0)),
            pl.BlockSpec((256, 16), lambda i: (0, 0)),
        ],
        out_specs=[
            pl.BlockSpec((BATCH, BLK), lambda i: (0, i)),
            pl.BlockSpec((BLK, 16), lambda i: (i, 0)),
        ],
        out_shape=[
            jax.ShapeDtypeStruct((BATCH, NPAD), jnp.int32),
            jax.ShapeDtypeStruct((NPAD, 16), jnp.int32),
        ],
    )(states_p, x_f, w_col, lut_p, wp)


@functools.partial(
    pl.kernel,
    out_type=jax.ShapeDtypeStruct((NPAD, BATCH), jnp.float32),
    mesh=plsc.VectorSubcoreMesh(core_axis_name="c", subcore_axis_name="s"),
    compiler_params=pltpu.CompilerParams(needs_layout_passes=False),
    scratch_types=[
        pltpu.VMEM((CH * K_MAX,), jnp.int32),
        pltpu.VMEM((CH * K_MAX,), jnp.int32),
        pltpu.VMEM((CH * K_MAX,), jnp.int32),
        pltpu.VMEM((CH * K_MAX, BATCH), jnp.int32),
        pltpu.VMEM((NT * 16,), jnp.int32),
        pltpu.VMEM((CH, BATCH), jnp.float32),
        pltpu.SemaphoreType.DMA,
    ],
)
def _sc_update(tbl_hbm, adjf_hbm, maskf_hbm, lutwf_hbm, out_hbm,
               adj_v, mask_v, idx_v, rows_v, lutw_v, out_v, sem):
    wid = lax.axis_index("c") * 16 + lax.axis_index("s")
    nbase = wid * NT
    pltpu.sync_copy(lutwf_hbm.at[pl.ds(nbase * 16, NT * 16)], lutw_v)

    def chunk_body(c, carry):
        node0 = nbase + c * CH
        pltpu.sync_copy(adjf_hbm.at[pl.ds(node0 * K_MAX, CH * K_MAX)], adj_v)
        pltpu.sync_copy(maskf_hbm.at[pl.ds(node0 * K_MAX, CH * K_MAX)], mask_v)
        for g in range(8):
            a = adj_v[pl.ds(g * 16, 16)]
            mk = mask_v[pl.ds(g * 16, 16)]
            idx_v[pl.ds(g * 16, 16)] = a * mk + ZROW * (1 - mk)
        pltpu.async_copy(tbl_hbm.at[idx_v], rows_v, sem).wait()

        def node_body(n, carry2):
            lbase = (c * CH + n) * 16
            for g in range(8):
                acc = rows_v[n * K_MAX, pl.ds(g * 16, 16)] << 7
                for k in range(1, K_MAX):
                    acc = acc | (rows_v[n * K_MAX + k, pl.ds(g * 16, 16)]
                                 << (7 - k))
                word = plsc.load_gather(lutw_v, [lbase + (acc >> 4)])
                bit = (word >> (acc & 15)) & 1
                out_v[n, pl.ds(g * 16, 16)] = bit.astype(jnp.float32)
            return carry2

        lax.fori_loop(0, CH, node_body, 0)
        pltpu.sync_copy(out_v, out_hbm.at[pl.ds(node0, CH)])
        return carry

    lax.fori_loop(0, NCHUNK, chunk_body, 0)


def _readout_body(bits_ref, w_ref, b_ref, out_ref):
    i = pl.program_id(0)
    part = jnp.dot(w_ref[...], bits_ref[...], preferred_element_type=jnp.float32)

    @pl.when(i == 0)
    def _init():
        out_ref[...] = jnp.zeros_like(out_ref)

    out_ref[...] += part

    @pl.when(i == NBLK - 1)
    def _fin():
        z = out_ref[...] + b_ref[...]
        out_ref[...] = 1.0 / (1.0 + jnp.exp(-z))


def _readout(bits, w_pad, b2d):
    return pl.pallas_call(
        _readout_body,
        grid=(NBLK,),
        in_specs=[
            pl.BlockSpec((BLK, BATCH), lambda i: (i, 0)),
            pl.BlockSpec((16, BLK), lambda i: (0, i)),
            pl.BlockSpec((16, BATCH), lambda i: (0, 0)),
        ],
        out_specs=pl.BlockSpec((16, BATCH), lambda i: (0, 0)),
        out_shape=jax.ShapeDtypeStruct((16, BATCH), jnp.float32),
    )(bits, w_pad, b2d)


def kernel(x, states, adj_list, adj_list_mask, lut, powers_of_2, w_in, W_out, b_out):
    del powers_of_2
    states_p = jnp.pad(states, ((0, 0), (0, NPAD - N_NODES)))
    adj_p = jnp.pad(adj_list, ((0, NPAD - N_NODES), (0, 0)))
    mask_p = jnp.pad(adj_list_mask, ((0, NPAD - N_NODES), (0, 0)))
    lut_p = jnp.pad(lut, ((0, NPAD - N_NODES), (0, 0)))
    x_f = x.astype(jnp.float32)
    w_col = w_in.reshape(INPUT_BITS, 1).astype(jnp.int32)
    cols = jnp.arange(256, dtype=jnp.int32)
    wp = ((cols[:, None] // 16 == jnp.arange(16, dtype=jnp.int32)[None, :])
          .astype(jnp.float32) * (2.0 ** (cols % 16).astype(jnp.float32))[:, None])
    w_pad = jnp.pad(W_out, ((0, 16 - N_OUT), (0, NPAD - N_NODES)))
    b2d = jnp.broadcast_to(
        jnp.pad(b_out, (0, 16 - N_OUT)).reshape(16, 1), (16, BATCH))

    pert, lutw = _prep(states_p, x_f, w_col, lut_p, wp)
    tbl = jnp.transpose(pert)
    bits = _sc_update(tbl, adj_p.reshape(-1), mask_p.reshape(-1),
                      lutw.reshape(-1))
    sig = _readout(bits, w_pad, b2d)
    return jnp.transpose(sig[:N_OUT, :])
